# baseline (speedup 1.0000x reference)
import jax
import jax.numpy as jnp
import numpy as np
from jax.experimental import pallas as pl
from jax.experimental.pallas import tpu as pltpu

_TEMPERATURE = 0.7
_TOP_P = 0.9
_B = 32
_H = 1024
_V = 100000
_TV = 2048
_NT = 49
_VPAD = _NT * _TV
_NITER = 24
_NEG = -1e30


def _np_threefry2x32(k1, k2, x0, x1):
    rots = ([13, 15, 26, 6], [17, 29, 16, 24])
    ks = (np.uint32(k1), np.uint32(k2),
          np.uint32(k1) ^ np.uint32(k2) ^ np.uint32(0x1BD11BDA))
    x0 = (x0 + ks[0]).astype(np.uint32)
    x1 = (x1 + ks[1]).astype(np.uint32)
    for i in range(5):
        for r in rots[i % 2]:
            x0 = (x0 + x1).astype(np.uint32)
            x1 = ((x1 << np.uint32(r)) | (x1 >> np.uint32(32 - r))).astype(np.uint32)
            x1 = x1 ^ x0
        x0 = (x0 + ks[(i + 1) % 3]).astype(np.uint32)
        x1 = (x1 + ks[(i + 2) % 3] + np.uint32(i + 1)).astype(np.uint32)
    return x0, x1


def _np_uniform_key42(shape):
    n = int(np.prod(shape))
    idx = np.arange(n, dtype=np.uint64)
    c_hi = (idx >> np.uint64(32)).astype(np.uint32)
    c_lo = (idx & np.uint64(0xFFFFFFFF)).astype(np.uint32)
    b1, b2 = _np_threefry2x32(0, 42, c_hi, c_lo)
    bits = b1 ^ b2
    float_bits = (bits >> np.uint32(9)) | np.uint32(0x3F800000)
    floats = float_bits.view(np.float32) - np.float32(1.0)
    tiny = np.float32(np.finfo(np.float32).tiny)
    span = np.float32(np.float32(1.0) - tiny)
    return np.maximum(tiny, floats * span + tiny).reshape(shape)


_U = np.full((_B, _VPAD), 0.5, np.float32)
_U[:, :_V] = _np_uniform_key42((_B, _V))


def _fused_kernel(h_ref, w_ref, u_ref, probs_ref, tok_ref,
                  l_s, e_s, m_s, s_s, lo_s, sk_s, bv_s, bi_s):
    p = pl.program_id(0)
    i = pl.program_id(1)

    @pl.when(p == 0)
    def _matmul_phase():
        acc = jax.lax.dot_general(
            h_ref[...], w_ref[...],
            dimension_numbers=(((1,), (1,)), ((), ())),
            preferred_element_type=jnp.float32,
        ) / _TEMPERATURE
        col = i * _TV + jax.lax.broadcasted_iota(jnp.int32, (_B, _TV), 1)
        tile = jnp.where(col < _V, acc, _NEG)
        l_s[:, pl.ds(i * _TV, _TV)] = tile
        tile_m = jnp.max(tile, axis=-1, keepdims=True)

        @pl.when(i == 0)
        def _():
            m_s[...] = tile_m
            s_s[...] = jnp.sum(jnp.exp(tile - tile_m), axis=-1, keepdims=True)

        @pl.when(i > 0)
        def _():
            m_old = m_s[...]
            m_new = jnp.maximum(m_old, tile_m)
            s_s[...] = (s_s[...] * jnp.exp(m_old - m_new)
                        + jnp.sum(jnp.exp(tile - m_new), axis=-1, keepdims=True))
            m_s[...] = m_new

    @pl.when(p == 1)
    def _topp_phase():
        @pl.when(i == 0)
        def _bisect():
            m = m_s[...]
            e_s[...] = jnp.exp(l_s[...] - m)
            e = e_s[...]
            target = jnp.float32(_TOP_P) * s_s[...]

            def body(_, carry):
                lo, hi = carry
                mid = 0.5 * (lo + hi)
                mass = jnp.sum(jnp.where(e > mid, e, 0.0), axis=-1, keepdims=True)
                above = mass > target
                return jnp.where(above, mid, lo), jnp.where(above, hi, mid)

            lo, _ = jax.lax.fori_loop(
                0, _NITER, body, (jnp.zeros_like(m), jnp.ones_like(m)))
            lo_s[...] = lo
            sk_s[...] = jnp.sum(jnp.where(e > lo, e, 0.0), axis=-1, keepdims=True)
            bv_s[...] = jnp.full_like(m, _NEG)
            bi_s[...] = jnp.zeros_like(m, dtype=jnp.int32)

        lo = lo_s[...]
        sk = sk_s[...]
        et = e_s[:, pl.ds(i * _TV, _TV)]
        lt = l_s[:, pl.ds(i * _TV, _TV)]
        keep_t = et > lo
        probs_ref[...] = jnp.where(keep_t, et / sk, 0.0)

        g = -jnp.log(-jnp.log(u_ref[...]))
        y = jnp.where(keep_t, lt, _NEG) + g
        tv = jnp.max(y, axis=-1, keepdims=True)
        ti = jnp.argmax(y, axis=-1, keepdims=True).astype(jnp.int32) + i * _TV
        upd = tv > bv_s[...]
        bi_s[...] = jnp.where(upd, ti, bi_s[...])
        bv_s[...] = jnp.where(upd, tv, bv_s[...])

        @pl.when(i == _NT - 1)
        def _():
            tok_ref[...] = bi_s[...]


def kernel(hidden_states, W):
    probs, tok = pl.pallas_call(
        _fused_kernel,
        grid=(2, _NT),
        in_specs=[
            pl.BlockSpec((_B, _H), lambda p, i: (0, 0)),
            pl.BlockSpec((_TV, _H), lambda p, i: (jnp.where(p == 0, i, _NT - 1), 0)),
            pl.BlockSpec((_B, _TV), lambda p, i: (0, jnp.where(p == 1, i, 0))),
        ],
        out_specs=[
            pl.BlockSpec((_B, _TV), lambda p, i: (0, jnp.where(p == 1, i, 0))),
            pl.BlockSpec((_B, 1), lambda p, i: (0, 0)),
        ],
        out_shape=[
            jax.ShapeDtypeStruct((_B, _V), jnp.float32),
            jax.ShapeDtypeStruct((_B, 1), jnp.int32),
        ],
        scratch_shapes=[
            pltpu.VMEM((_B, _VPAD), jnp.float32),
            pltpu.VMEM((_B, _VPAD), jnp.float32),
            pltpu.VMEM((_B, 1), jnp.float32),
            pltpu.VMEM((_B, 1), jnp.float32),
            pltpu.VMEM((_B, 1), jnp.float32),
            pltpu.VMEM((_B, 1), jnp.float32),
            pltpu.VMEM((_B, 1), jnp.float32),
            pltpu.VMEM((_B, 1), jnp.int32),
        ],
    )(hidden_states, W, jnp.asarray(_U))

    return probs, tok.reshape(-1)


# speedup vs baseline: 1.1357x; 1.1357x over previous
import jax
import jax.numpy as jnp
import numpy as np
from jax.experimental import pallas as pl

_TEMPERATURE = 0.7
_TOP_P = 0.9
_B = 32
_H = 1024
_V = 100000
_TV = 2048
_VPAD = 100352
_RB = 16
_NITER = 20
_NEG = -1e30


def _np_threefry2x32(k1, k2, x0, x1):
    rots = ([13, 15, 26, 6], [17, 29, 16, 24])
    ks = (np.uint32(k1), np.uint32(k2),
          np.uint32(k1) ^ np.uint32(k2) ^ np.uint32(0x1BD11BDA))
    x0 = (x0 + ks[0]).astype(np.uint32)
    x1 = (x1 + ks[1]).astype(np.uint32)
    for i in range(5):
        for r in rots[i % 2]:
            x0 = (x0 + x1).astype(np.uint32)
            x1 = ((x1 << np.uint32(r)) | (x1 >> np.uint32(32 - r))).astype(np.uint32)
            x1 = x1 ^ x0
        x0 = (x0 + ks[(i + 1) % 3]).astype(np.uint32)
        x1 = (x1 + ks[(i + 2) % 3] + np.uint32(i + 1)).astype(np.uint32)
    return x0, x1


def _np_uniform_key42(shape):
    n = int(np.prod(shape))
    idx = np.arange(n, dtype=np.uint64)
    c_hi = (idx >> np.uint64(32)).astype(np.uint32)
    c_lo = (idx & np.uint64(0xFFFFFFFF)).astype(np.uint32)
    b1, b2 = _np_threefry2x32(0, 42, c_hi, c_lo)
    bits = b1 ^ b2
    float_bits = (bits >> np.uint32(9)) | np.uint32(0x3F800000)
    floats = float_bits.view(np.float32) - np.float32(1.0)
    tiny = np.float32(np.finfo(np.float32).tiny)
    span = np.float32(np.float32(1.0) - tiny)
    return np.maximum(tiny, floats * span + tiny).reshape(shape)


_U = np.full((_B, _VPAD), 0.5, np.float32)
_U[:, :_V] = _np_uniform_key42((_B, _V))


def _matmul_kernel(h_ref, w_ref, out_ref):
    i = pl.program_id(0)
    acc = jax.lax.dot_general(
        h_ref[...], w_ref[...],
        dimension_numbers=(((1,), (1,)), ((), ())),
        preferred_element_type=jnp.float32,
    ) / _TEMPERATURE
    col = i * _TV + jax.lax.broadcasted_iota(jnp.int32, (_B, _TV), 1)
    out_ref[...] = jnp.where(col < _V, acc, _NEG)


def _topp_kernel(l_ref, u_ref, probs_ref, tok_ref):
    l = l_ref[...]
    m = jnp.max(l, axis=-1, keepdims=True)
    e = jnp.exp(l - m)
    s_full = jnp.sum(e, axis=-1, keepdims=True)
    target = jnp.float32(_TOP_P) * s_full

    def body(_, carry):
        lo, hi = carry
        mid = 0.5 * (lo + hi)
        mass = jnp.sum(jnp.where(e > mid, e, 0.0), axis=-1, keepdims=True)
        above = mass > target
        return jnp.where(above, mid, lo), jnp.where(above, hi, mid)

    lo, _ = jax.lax.fori_loop(
        0, _NITER, body, (jnp.zeros_like(m), jnp.ones_like(m)))

    keep = e > lo
    s_keep = jnp.sum(jnp.where(keep, e, 0.0), axis=-1, keepdims=True)
    probs = jnp.where(keep, e / s_keep, 0.0)
    probs_ref[...] = probs[:, :_V]

    g = -jnp.log(-jnp.log(u_ref[...]))
    y = jnp.where(keep, l, _NEG) + g
    tok_ref[...] = jnp.argmax(y, axis=-1, keepdims=True).astype(jnp.int32)


def kernel(hidden_states, W):
    logits = pl.pallas_call(
        _matmul_kernel,
        grid=(_VPAD // _TV,),
        in_specs=[
            pl.BlockSpec((_B, _H), lambda i: (0, 0)),
            pl.BlockSpec((_TV, _H), lambda i: (i, 0)),
        ],
        out_specs=pl.BlockSpec((_B, _TV), lambda i: (0, i)),
        out_shape=jax.ShapeDtypeStruct((_B, _VPAD), jnp.float32),
    )(hidden_states, W)

    probs, tok = pl.pallas_call(
        _topp_kernel,
        grid=(_B // _RB,),
        in_specs=[
            pl.BlockSpec((_RB, _VPAD), lambda i: (i, 0)),
            pl.BlockSpec((_RB, _VPAD), lambda i: (i, 0)),
        ],
        out_specs=[
            pl.BlockSpec((_RB, _V), lambda i: (i, 0)),
            pl.BlockSpec((_RB, 1), lambda i: (i, 0)),
        ],
        out_shape=[
            jax.ShapeDtypeStruct((_B, _V), jnp.float32),
            jax.ShapeDtypeStruct((_B, 1), jnp.int32),
        ],
    )(logits, jnp.asarray(_U))

    return probs, tok.reshape(-1)
